# baseline (speedup 1.0000x reference)
import functools
import jax
import jax.numpy as jnp
from jax.experimental import pallas as pl
from jax.experimental.pallas import tpu as pltpu

_GW = 8


def _gather8_kernel(tok_ref, *refs):
    out_ref = refs[_GW]
    for k in range(_GW):
        out_ref[0, k, :] = refs[k][0, 0, :]


def _prologue_kernel(h_ref, Wq_ref, col_id_ref, Wk_in_ref, Wv_in_ref,
                     w_decay_ref, b_decay_ref, bias_ref, Wk_out_ref,
                     mq_ref, ip_ref,
                     v_ref, cntN_ref, g_ref, u_ref, a_ref,
                     *, B, N, D_s, H, Dq, N_in):
    h = h_ref[...]
    q = jnp.dot(h, Wq_ref[...], preferred_element_type=jnp.float32)
    iota_n = jax.lax.broadcasted_iota(jnp.int32, (N_in, N), 1)
    P = (iota_n == ip_ref[...]).astype(jnp.float32)
    in_ids = jnp.dot(P, col_id_ref[...], preferred_element_type=jnp.float32)
    keys = jnp.dot(in_ids, Wk_in_ref[...], preferred_element_type=jnp.float32)
    inv_sqrt_dq = 1.0 / (Dq ** 0.5)
    ji = jax.lax.broadcasted_iota(jnp.int32, (B, N_in), 1)
    cnt_in = jnp.zeros((B, N_in), dtype=jnp.float32)
    for hh in range(H):
        qh = q[:, hh * Dq:(hh + 1) * Dq]
        sc = jax.lax.dot_general(qh, keys, (((1,), (1,)), ((), ())),
                                 preferred_element_type=jnp.float32) * inv_sqrt_dq
        sc = sc + bias_ref[hh:hh + 1, :]
        mx = jnp.max(sc, axis=1, keepdims=True)
        sel = jnp.where(sc == mx, ji, N_in)
        jloc = jnp.min(sel, axis=1, keepdims=True)
        cnt_in = cnt_in + (ji == jloc).astype(jnp.float32)
    cntN_ref[...] = jnp.dot(cnt_in, P, preferred_element_type=jnp.float32)
    v = jnp.dot(h, Wv_in_ref[...], preferred_element_type=jnp.float32)
    v_ref[...] = v
    a_ref[...] = jax.nn.sigmoid(
        jnp.dot(col_id_ref[...], w_decay_ref[...],
                preferred_element_type=jnp.float32) + b_decay_ref[...])
    u = jax.lax.dot_general(mq_ref[...], Wk_out_ref[...], (((1,), (1,)), ((), ())),
                            preferred_element_type=jnp.float32)
    u_ref[...] = u
    g_ref[...] = jax.lax.dot_general(v, u, (((1,), (1,)), ((), ())),
                                     preferred_element_type=jnp.float32)


def _main_kernel(s_ref, u_ref, a_ref, cnt_ref, g_ref, v_ref, o_ref,
                 macc, mmax, lsum, cwacc, *, nsteps, D_s):
    j = pl.program_id(1)
    sb = s_ref[...]
    u3 = jnp.reshape(u_ref[...], (1, 1, D_s))
    t = jnp.sum(sb * u3, axis=2)
    a = a_ref[...]
    cnt = cnt_ref[...]
    logit = (t * a + cnt * g_ref[...]) * (1.0 / (D_s ** 0.5))
    bm = jnp.max(logit, axis=1, keepdims=True)

    @pl.when(j == 0)
    def _init():
        mmax[...] = jnp.full_like(mmax[...], -jnp.inf)
        lsum[...] = jnp.zeros_like(lsum[...])
        cwacc[...] = jnp.zeros_like(cwacc[...])
        macc[...] = jnp.zeros_like(macc[...])

    newm = jnp.maximum(mmax[...], bm)
    scale = jnp.exp(mmax[...] - newm)
    e = jnp.exp(logit - newm)
    w = e * a
    part = jnp.einsum('bn,bnd->bd', w, sb, preferred_element_type=jnp.float32)
    macc[...] = macc[...] * scale + part
    lsum[...] = lsum[...] * scale + jnp.sum(e, axis=1, keepdims=True)
    cwacc[...] = cwacc[...] * scale + jnp.sum(e * cnt, axis=1, keepdims=True)
    mmax[...] = newm

    @pl.when(j == nsteps - 1)
    def _fin():
        inv_l = 1.0 / lsum[...]
        o_ref[...] = macc[...] * inv_l + (cwacc[...] * inv_l) * v_ref[...]


def _logits_kernel(m_ref, Wv_ref, te_ref, out_ref, motor_s):
    @pl.when(pl.program_id(0) == 0)
    def _motor():
        motor0 = jnp.dot(m_ref[...], Wv_ref[...],
                         preferred_element_type=jnp.float32)
        ms = jnp.mean(motor0 * motor0, axis=1, keepdims=True)
        motor_s[...] = motor0 * jax.lax.rsqrt(ms + 1e-6)

    out_ref[...] = jax.lax.dot_general(
        motor_s[...], te_ref[...], (((1,), (1,)), ((), ())),
        preferred_element_type=jnp.float32)


def kernel(token_id, s, tok_emb, Wq, col_id, Wk_in, Wv_in, w_decay, b_decay,
           input_E_bias, Wk_out, Wv_out, motor_query, input_positions):
    B, N, D_s = s.shape
    V = tok_emb.shape[0]
    H, N_in = input_E_bias.shape
    Dq = Wk_in.shape[1]
    f32 = jnp.float32

    te3 = tok_emb.reshape(V, 1, D_s)

    def _mk_map(k):
        return lambda i, tok: (tok[_GW * i + k], 0, 0)

    h = pl.pallas_call(
        _gather8_kernel,
        grid_spec=pltpu.PrefetchScalarGridSpec(
            num_scalar_prefetch=1,
            grid=(B // _GW,),
            in_specs=[pl.BlockSpec((1, 1, D_s), _mk_map(k)) for k in range(_GW)],
            out_specs=pl.BlockSpec((1, _GW, D_s), lambda i, tok: (i, 0, 0)),
        ),
        out_shape=jax.ShapeDtypeStruct((B // _GW, _GW, D_s), f32),
    )(token_id.astype(jnp.int32), *([te3] * _GW))
    h = h.reshape(B, D_s)

    mq2 = motor_query.reshape(1, D_s)
    bd2 = b_decay.reshape(1, 1)
    ip2 = input_positions.astype(jnp.int32).reshape(N_in, 1)
    v, cntN, g, u_row, a_col = pl.pallas_call(
        functools.partial(_prologue_kernel, B=B, N=N, D_s=D_s, H=H, Dq=Dq,
                          N_in=N_in),
        out_shape=(
            jax.ShapeDtypeStruct((B, D_s), f32),
            jax.ShapeDtypeStruct((B, N), f32),
            jax.ShapeDtypeStruct((B, 1), f32),
            jax.ShapeDtypeStruct((1, D_s), f32),
            jax.ShapeDtypeStruct((N, 1), f32),
        ),
    )(h, Wq, col_id, Wk_in, Wv_in, w_decay, bd2, input_E_bias, Wk_out,
      mq2, ip2)
    a_row = a_col.reshape(1, N)

    bb, nb = 8, 512
    nsteps = N // nb
    m_tot = pl.pallas_call(
        functools.partial(_main_kernel, nsteps=nsteps, D_s=D_s),
        grid=(B // bb, nsteps),
        in_specs=[
            pl.BlockSpec((bb, nb, D_s), lambda i, j: (i, j, 0)),
            pl.BlockSpec((1, D_s), lambda i, j: (0, 0)),
            pl.BlockSpec((1, nb), lambda i, j: (0, j)),
            pl.BlockSpec((bb, nb), lambda i, j: (i, j)),
            pl.BlockSpec((bb, 1), lambda i, j: (i, 0)),
            pl.BlockSpec((bb, D_s), lambda i, j: (i, 0)),
        ],
        out_specs=pl.BlockSpec((bb, D_s), lambda i, j: (i, 0)),
        out_shape=jax.ShapeDtypeStruct((B, D_s), f32),
        scratch_shapes=[
            pltpu.VMEM((bb, D_s), f32),
            pltpu.VMEM((bb, 1), f32),
            pltpu.VMEM((bb, 1), f32),
            pltpu.VMEM((bb, 1), f32),
        ],
    )(s, u_row, a_row, cntN, g, v)

    vb = 2048
    logits = pl.pallas_call(
        _logits_kernel,
        grid=(V // vb,),
        in_specs=[
            pl.BlockSpec((B, D_s), lambda j: (0, 0)),
            pl.BlockSpec((D_s, D_s), lambda j: (0, 0)),
            pl.BlockSpec((vb, D_s), lambda j: (j, 0)),
        ],
        out_specs=pl.BlockSpec((B, vb), lambda j: (0, j)),
        out_shape=jax.ShapeDtypeStruct((B, V), f32),
        scratch_shapes=[pltpu.VMEM((B, D_s), f32)],
    )(m_tot, Wv_out, tok_emb)
    return logits
